# baseline (speedup 1.0000x reference)
import functools

import jax
import jax.numpy as jnp
from jax import lax
from jax.experimental import pallas as pl
from jax.experimental.pallas import tpu as pltpu
from jax.experimental.pallas import tpu_sc as plsc


def _make_avg_sc(B, C, V, D, DP):
    info = plsc.get_sparse_core_info()
    NC, NS, L = info.num_cores, info.num_subcores, info.num_lanes
    NW = NC * NS
    assert B % NW == 0 and D % L == 0
    b_per_w = B // NW
    n_idx = b_per_w * C
    n_vec = D // L
    NCH = 4
    rows_per_ch = b_per_w // NCH
    idx_per_ch = rows_per_ch * C
    mesh = plsc.VectorSubcoreMesh(core_axis_name="c", subcore_axis_name="s")

    def slices(total):
        out, off = [], 0
        while off < total:
            ln = min(128, total - off)
            out.append((off, ln))
            off += ln
        return out

    @functools.partial(
        pl.kernel,
        mesh=mesh,
        compiler_params=pltpu.CompilerParams(use_tc_tiling_on_sc=False),
        out_type=jax.ShapeDtypeStruct((B, D), jnp.float32),
        scratch_types=[
            pltpu.VMEM((n_idx,), jnp.int32),
            pltpu.VMEM((idx_per_ch, DP), jnp.float32),
            pltpu.VMEM((idx_per_ch, DP), jnp.float32),
            pltpu.VMEM((b_per_w, D), jnp.float32),
            pltpu.SemaphoreType.DMA,
        ],
    )
    def avg_kernel(ctx_hbm, table_hbm, out_hbm, idx_v, rows_a, rows_b, avg_v, sem):
        wid = lax.axis_index("s") * NC + lax.axis_index("c")
        base = wid * b_per_w
        bufs = [rows_a, rows_b]
        pltpu.sync_copy(ctx_hbm.at[pl.ds(base * C, n_idx)], idx_v)

        def fire(ch):
            buf = bufs[ch % 2]
            return [
                pltpu.async_copy(
                    table_hbm.at[idx_v.at[pl.ds(ch * idx_per_ch + o, ln)]],
                    buf.at[pl.ds(o, ln)],
                    sem,
                )
                for o, ln in slices(idx_per_ch)
            ]

        inv_c = jnp.float32(1.0 / C)

        def reduce_chunk(ch):
            buf = bufs[ch % 2]

            def row_body(bi, carry):
                rb = bi * C
                unroll = 2
                assert C % unroll == 0

                def ctx_body(ci, accs):
                    for u in range(unroll):
                        accs = tuple(
                            accs[v] + buf[rb + ci * unroll + u, pl.ds(v * L, L)]
                            for v in range(n_vec)
                        )
                    return accs

                accs = lax.fori_loop(
                    0, C // unroll, ctx_body,
                    tuple(jnp.zeros((L,), jnp.float32) for _ in range(n_vec)),
                )
                for v in range(n_vec):
                    avg_v[ch * rows_per_ch + bi, pl.ds(v * L, L)] = accs[v] * inv_c
                return carry

            lax.fori_loop(0, rows_per_ch, row_body, jnp.int32(0))

        pending = fire(0)
        for ch in range(NCH):
            for h in pending:
                h.wait()
            pending = fire(ch + 1) if ch + 1 < NCH else []
            reduce_chunk(ch)

        pltpu.sync_copy(avg_v, out_hbm.at[pl.ds(base, b_per_w)])

    return avg_kernel


def _logits_t_tc(avg, Wt, b2, BN=4096):
    B, D = avg.shape
    V = Wt.shape[1]

    def mm(avg_ref, w_ref, b_ref, out_ref):
        acc = lax.dot_general(
            w_ref[...], avg_ref[...],
            (((0,), (1,)), ((), ())),
            preferred_element_type=jnp.float32,
        )
        ones = jnp.ones((B, 1), jnp.float32)
        bias = lax.dot_general(
            b_ref[...], ones,
            (((0,), (1,)), ((), ())),
            preferred_element_type=jnp.float32,
        )
        out_ref[...] = acc + bias

    return pl.pallas_call(
        mm,
        grid=(pl.cdiv(V, BN),),
        in_specs=[
            pl.BlockSpec((B, D), lambda i: (0, 0)),
            pl.BlockSpec((D, BN), lambda i: (0, i)),
            pl.BlockSpec((1, BN), lambda i: (0, i)),
        ],
        out_specs=pl.BlockSpec((BN, B), lambda i: (i, 0)),
        out_shape=jax.ShapeDtypeStruct((V, B), jnp.float32),
    )(avg, Wt, b2)


def kernel(contexts, table, W, b):
    B, C = contexts.shape
    V, D = table.shape
    DP = 128
    table_pad = jnp.pad(table, ((0, 0), (0, DP - D)))
    avg = _make_avg_sc(B, C, V, D, DP)(contexts.reshape(-1), table_pad)
    logits_t = _logits_t_tc(avg, W.T, b.reshape(1, V))
    return logits_t.T


# speedup vs baseline: 1.0224x; 1.0224x over previous
import functools

import jax
import jax.numpy as jnp
from jax import lax
from jax.experimental import pallas as pl
from jax.experimental.pallas import tpu as pltpu
from jax.experimental.pallas import tpu_sc as plsc


def _make_avg_sc(B, C, V, D, DP):
    info = plsc.get_sparse_core_info()
    NC, NS, L = info.num_cores, info.num_subcores, info.num_lanes
    NW = NC * NS
    assert B % NW == 0 and D % L == 0
    b_per_w = B // NW
    n_idx = b_per_w * C
    n_vec = D // L
    NCH = 4
    rows_per_ch = b_per_w // NCH
    idx_per_ch = rows_per_ch * C
    mesh = plsc.VectorSubcoreMesh(core_axis_name="c", subcore_axis_name="s")

    def slices(total):
        out, off = [], 0
        while off < total:
            ln = min(128, total - off)
            out.append((off, ln))
            off += ln
        return out

    @functools.partial(
        pl.kernel,
        mesh=mesh,
        compiler_params=pltpu.CompilerParams(use_tc_tiling_on_sc=False),
        out_type=jax.ShapeDtypeStruct((B, D), jnp.float32),
        scratch_types=[
            pltpu.VMEM((n_idx,), jnp.int32),
            pltpu.VMEM((idx_per_ch, D), jnp.float32),
            pltpu.VMEM((idx_per_ch, D), jnp.float32),
            pltpu.VMEM((b_per_w, D), jnp.float32),
            pltpu.SemaphoreType.DMA,
        ],
    )
    def avg_kernel(ctx_hbm, table_hbm, out_hbm, idx_v, rows_a, rows_b, avg_v, sem):
        wid = lax.axis_index("s") * NC + lax.axis_index("c")
        base = wid * b_per_w
        bufs = [rows_a, rows_b]
        pltpu.sync_copy(ctx_hbm.at[pl.ds(base * C, n_idx)], idx_v)
        mul = DP // D

        def dbl_body(i, carry):
            idx_v[pl.ds(i * L, L)] = idx_v[pl.ds(i * L, L)] * mul
            return carry

        lax.fori_loop(0, n_idx // L, dbl_body, jnp.int32(0))

        def fire(ch):
            buf = bufs[ch % 2]
            return [
                pltpu.async_copy(
                    table_hbm.at[idx_v.at[pl.ds(ch * idx_per_ch + o, ln)]],
                    buf.at[pl.ds(o, ln)],
                    sem,
                )
                for o, ln in slices(idx_per_ch)
            ]

        inv_c = jnp.float32(1.0 / C)

        def reduce_chunk(ch):
            buf = bufs[ch % 2]

            def row_body(bi, carry):
                rb = bi * C
                unroll = 2
                assert C % unroll == 0

                def ctx_body(ci, accs):
                    for u in range(unroll):
                        accs = tuple(
                            accs[v] + buf[rb + ci * unroll + u, pl.ds(v * L, L)]
                            for v in range(n_vec)
                        )
                    return accs

                accs = lax.fori_loop(
                    0, C // unroll, ctx_body,
                    tuple(jnp.zeros((L,), jnp.float32) for _ in range(n_vec)),
                )
                for v in range(n_vec):
                    avg_v[ch * rows_per_ch + bi, pl.ds(v * L, L)] = accs[v] * inv_c
                return carry

            lax.fori_loop(0, rows_per_ch, row_body, jnp.int32(0))

        pending = fire(0)
        for ch in range(NCH):
            for h in pending:
                h.wait()
            pending = fire(ch + 1) if ch + 1 < NCH else []
            reduce_chunk(ch)

        pltpu.sync_copy(avg_v, out_hbm.at[pl.ds(base, b_per_w)])

    return avg_kernel


def _logits_t_tc(avg, Wt, b2, BN=4096):
    B, D = avg.shape
    V = Wt.shape[1]

    def mm(avg_ref, w_ref, b_ref, out_ref):
        acc = lax.dot_general(
            w_ref[...], avg_ref[...],
            (((0,), (1,)), ((), ())),
            preferred_element_type=jnp.float32,
        )
        ones = jnp.ones((B, 1), jnp.float32)
        bias = lax.dot_general(
            b_ref[...], ones,
            (((0,), (1,)), ((), ())),
            preferred_element_type=jnp.float32,
        )
        out_ref[...] = acc + bias

    return pl.pallas_call(
        mm,
        grid=(pl.cdiv(V, BN),),
        in_specs=[
            pl.BlockSpec((B, D), lambda i: (0, 0)),
            pl.BlockSpec((D, BN), lambda i: (0, i)),
            pl.BlockSpec((1, BN), lambda i: (0, i)),
        ],
        out_specs=pl.BlockSpec((BN, B), lambda i: (i, 0)),
        out_shape=jax.ShapeDtypeStruct((V, B), jnp.float32),
    )(avg, Wt, b2)


def kernel(contexts, table, W, b):
    B, C = contexts.shape
    V, D = table.shape
    DP = 128
    table_pad = jnp.pad(table, ((0, 0), (0, DP - D)))
    table_rows = table_pad.reshape(V * (DP // D), D)
    avg = _make_avg_sc(B, C, V, D, DP)(contexts.reshape(-1), table_rows)
    logits_t = _logits_t_tc(avg, W.T, b.reshape(1, V))
    return logits_t.T
